# baseline (speedup 1.0000x reference)
import functools

import jax
import jax.numpy as jnp
from jax import lax
from jax.experimental import pallas as pl
from jax.experimental.pallas import tpu as pltpu
from jax.experimental.pallas import tpu_sc as plsc

_ROWS_PER_BLOCK = 128
_SC_RB = 64


def _tc_kernel(new_p, new_z, new_b, old_p, old_z, old_b,
               out_p, out_z, out_b):
    for new, old, out in ((new_p, old_p, out_p),
                          (new_z, old_z, out_z),
                          (new_b, old_b, out_b)):
        w = new.shape[-1]
        out[..., :w] = new[...]
        out[..., w:] = old[...]


def _make_sc_copy(R, W, Wsub, dtype):
    info = plsc.get_sparse_core_info()
    NW = info.num_cores * info.num_subcores
    rows_per_w = R // NW
    n_iter = rows_per_w // _SC_RB
    mesh = plsc.VectorSubcoreMesh(core_axis_name="c", subcore_axis_name="s")

    @functools.partial(
        pl.kernel, mesh=mesh,
        out_type=jax.ShapeDtypeStruct((R, W), dtype),
        scratch_types=[
            pltpu.VMEM((_SC_RB, Wsub), dtype),
            pltpu.VMEM((_SC_RB, W - Wsub), dtype),
            pltpu.SemaphoreType.DMA,
            pltpu.SemaphoreType.DMA,
        ],
    )
    def sc_copy(new_hbm, old_hbm, out_hbm, nb, ob, sem_in, sem_out):
        wid = lax.axis_index("s") * info.num_cores + lax.axis_index("c")
        base = wid * rows_per_w

        def body(i, _):
            r0 = base + i * _SC_RB
            cin1 = pltpu.make_async_copy(
                new_hbm.at[pl.ds(r0, _SC_RB), :], nb, sem_in)
            cin2 = pltpu.make_async_copy(
                old_hbm.at[pl.ds(r0, _SC_RB), pl.ds(Wsub, W - Wsub)], ob,
                sem_in)
            cin1.start(); cin2.start()
            cin1.wait(); cin2.wait()
            co1 = pltpu.make_async_copy(
                nb, out_hbm.at[pl.ds(r0, _SC_RB), pl.ds(0, Wsub)], sem_out)
            co2 = pltpu.make_async_copy(
                ob, out_hbm.at[pl.ds(r0, _SC_RB), pl.ds(Wsub, W - Wsub)],
                sem_out)
            co1.start(); co2.start()
            co1.wait(); co2.wait()
            return ()

        lax.fori_loop(0, n_iter, body, ())

    return sc_copy


def kernel(pix_to_face, zbuf, bary_coords, dists, indices,
           new_pix_to_face, new_zbuf, new_bary_coords, new_dists):
    N, H, W, K = pix_to_face.shape
    Wsub = new_pix_to_face.shape[2]
    RB = _ROWS_PER_BLOCK

    t4 = lambda x: jnp.transpose(x, (0, 1, 3, 2))
    t5 = lambda x: jnp.transpose(x, (0, 1, 4, 3, 2))

    old_p, old_z = t4(pix_to_face), t4(zbuf)
    old_b = t5(bary_coords)
    new_p, new_z = t4(new_pix_to_face), t4(new_zbuf)
    new_b = t5(new_bary_coords)

    R2 = N * H * K
    old_d2 = t4(dists).reshape(R2, W)
    new_d2 = t4(new_dists).reshape(R2, Wsub)
    sc_copy = _make_sc_copy(R2, W, Wsub, dists.dtype)
    out_d2 = sc_copy(new_d2, old_d2)

    new4 = pl.BlockSpec((1, RB, K, Wsub), lambda n, h: (n, h, 0, 0))
    tail4 = pl.BlockSpec((1, RB, K, Wsub), lambda n, h: (n, h, 0, 1))
    out4 = pl.BlockSpec((1, RB, K, W), lambda n, h: (n, h, 0, 0))
    new5 = pl.BlockSpec((1, RB, 3, K, Wsub), lambda n, h: (n, h, 0, 0, 0))
    tail5 = pl.BlockSpec((1, RB, 3, K, Wsub), lambda n, h: (n, h, 0, 0, 1))
    out5 = pl.BlockSpec((1, RB, 3, K, W), lambda n, h: (n, h, 0, 0, 0))

    out_p, out_z, out_b = pl.pallas_call(
        _tc_kernel,
        grid=(N, H // RB),
        in_specs=[new4, new4, new5, tail4, tail4, tail5],
        out_specs=[out4, out4, out5],
        out_shape=[
            jax.ShapeDtypeStruct((N, H, K, W), pix_to_face.dtype),
            jax.ShapeDtypeStruct((N, H, K, W), zbuf.dtype),
            jax.ShapeDtypeStruct((N, H, 3, K, W), bary_coords.dtype),
        ],
        compiler_params=pltpu.CompilerParams(
            dimension_semantics=("arbitrary", "arbitrary"),
        ),
    )(new_p, new_z, new_b, old_p, old_z, old_b)

    out_d = out_d2.reshape(N, H, K, W)
    return (jnp.transpose(out_p, (0, 1, 3, 2)),
            jnp.transpose(out_z, (0, 1, 3, 2)),
            jnp.transpose(out_b, (0, 1, 4, 3, 2)),
            jnp.transpose(out_d, (0, 1, 3, 2)))
